# baseline (speedup 1.0000x reference)
import jax
import jax.numpy as jnp
from jax import lax
from jax.experimental import pallas as pl
from jax.experimental.pallas import tpu as pltpu
from jax.experimental.pallas import tpu_sc as plsc

N_TOKEN = 1000000
EMB_DIM = 64
BATCH = 16384
HIST = 20

NC = 2
NS = 16
NW = NC * NS

B = BATCH * HIST
B_PER_W = B // NW
CHUNK = 128
N_CHUNKS = B_PER_W // CHUNK
NBUF = 8
T = N_CHUNKS // NBUF

VB = 32768
NVB = (N_TOKEN + VB - 1) // VB
N2 = NVB * VB


def _tr_body(wt_ref, out_ref):
    blk = wt_ref[...]
    tr = jnp.transpose(blk, (1, 0))
    out_ref[:, 0:EMB_DIM] = tr[: VB // 2, :]
    out_ref[:, EMB_DIM:128] = tr[VB // 2 :, :]


def _transpose_tc(wT):
    return pl.pallas_call(
        _tr_body,
        grid=(NVB,),
        in_specs=[pl.BlockSpec((EMB_DIM, VB), lambda j: (0, j))],
        out_specs=pl.BlockSpec((VB // 2, 128), lambda j: (j, 0)),
        out_shape=jax.ShapeDtypeStruct((N2 // 2, 128), jnp.float32),
    )(wT)


BLK_B = 1024
NBB = BATCH // BLK_B
RPB = BLK_B * HIST * EMB_DIM // 128


def _ot_body(in_ref, out_ref):
    whole = in_ref[...]
    w3 = whole.reshape(BLK_B, HIST * EMB_DIM // 128, 128)
    for h in range(HIST):
        lane = (h % 2) * EMB_DIM
        part = w3[:, h // 2, lane : lane + EMB_DIM]
        out_ref[h] = jnp.transpose(part, (1, 0))


def _out_transpose(o2):
    return pl.pallas_call(
        _ot_body,
        grid=(NBB,),
        in_specs=[pl.BlockSpec((RPB, 128), lambda i: (i, 0))],
        out_specs=pl.BlockSpec((HIST, EMB_DIM, BLK_B), lambda i: (0, 0, i)),
        out_shape=jax.ShapeDtypeStruct((HIST, EMB_DIM, BATCH), jnp.float32),
    )(o2)


def _emb_body(idx_hbm, table_hbm, out_hbm, idx_v, rows_v, sems):
    wid = lax.axis_index("s") * NC + lax.axis_index("c")
    base = wid * B_PER_W

    pltpu.sync_copy(idx_hbm.at[wid], idx_v)

    def remap(i, carry):
        r = i // (CHUNK // 16)
        c = (i % (CHUNK // 16)) * 16
        v = idx_v[r, pl.ds(c, 16)]
        rp = v & (VB - 1)
        lo = v + rp
        row = jnp.where(rp < VB // 2, lo, lo - (VB - 1))
        idx_v[r, pl.ds(c, 16)] = row
        return carry

    lax.fori_loop(0, N_CHUNKS * (CHUNK // 16), remap, 0)

    def start(b, j):
        pltpu.async_copy(table_hbm.at[idx_v.at[j]], rows_v.at[b], sems.at[b])

    def finish(b, j):
        pltpu.make_async_copy(
            table_hbm.at[idx_v.at[j]], rows_v.at[b], sems.at[b]
        ).wait()
        pltpu.sync_copy(rows_v.at[b], out_hbm.at[pl.ds(base + j * CHUNK, CHUNK)])

    for b in range(NBUF):
        start(b, b)

    def body(it, carry):
        for b in range(NBUF):
            j = it * NBUF + b
            finish(b, j)
            start(b, j + NBUF)
        return carry

    lax.fori_loop(0, T - 1, body, 0)

    for b in range(NBUF):
        finish(b, (T - 1) * NBUF + b)


@jax.jit
def _emb_call(x_flat, table):
    mesh = plsc.VectorSubcoreMesh(core_axis_name="c", subcore_axis_name="s")
    kern = pl.kernel(
        _emb_body,
        out_type=jax.ShapeDtypeStruct((B, EMB_DIM), jnp.float32),
        mesh=mesh,
        scratch_types=[
            pltpu.VMEM((N_CHUNKS, CHUNK), jnp.int32),
            pltpu.VMEM((NBUF, CHUNK, EMB_DIM), jnp.float32),
            pltpu.SemaphoreType.DMA((NBUF,)),
        ],
        compiler_params=pltpu.CompilerParams(use_tc_tiling_on_sc=False),
    )
    return kern(x_flat, table)


def kernel(x, weight):
    w2 = _transpose_tc(weight.T)
    table = w2.reshape(N2, EMB_DIM)
    x_flat = x.reshape(NW, N_CHUNKS, CHUNK).astype(jnp.int32)
    out = _emb_call(x_flat, table)
    o2 = out.reshape(B * EMB_DIM // 128, 128)
    outT = _out_transpose(o2)
    return jnp.transpose(outT, (2, 0, 1))


# speedup vs baseline: 1.0025x; 1.0025x over previous
import jax
import jax.numpy as jnp
from jax import lax
from jax.experimental import pallas as pl
from jax.experimental.pallas import tpu as pltpu
from jax.experimental.pallas import tpu_sc as plsc

N_TOKEN = 1000000
EMB_DIM = 64
BATCH = 16384
HIST = 20

NC = 2
NS = 16
NW = NC * NS

B = BATCH * HIST
B_PER_W = B // NW
CHUNK = 128
N_CHUNKS = B_PER_W // CHUNK
NBUF = 8
T = N_CHUNKS // NBUF

VB = 32768
NVB = (N_TOKEN + VB - 1) // VB
N2 = NVB * VB


def _tr_body(wt_ref, out_ref):
    blk = wt_ref[...]
    tr = jnp.transpose(blk, (1, 0))
    out_ref[:, 0:EMB_DIM] = tr[: VB // 2, :]
    out_ref[:, EMB_DIM:128] = tr[VB // 2 :, :]


def _transpose_tc(wT):
    return pl.pallas_call(
        _tr_body,
        grid=(NVB,),
        in_specs=[pl.BlockSpec((EMB_DIM, VB), lambda j: (0, j))],
        out_specs=pl.BlockSpec((VB // 2, 128), lambda j: (j, 0)),
        out_shape=jax.ShapeDtypeStruct((N2 // 2, 128), jnp.float32),
    )(wT)


BLK_B = 512
NBB = BATCH // BLK_B
RPB = BLK_B * HIST * EMB_DIM // 128


def _ot_body(in_ref, out_ref):
    whole = in_ref[...]
    w3 = whole.reshape(BLK_B, HIST * EMB_DIM // 128, 128)
    for h in range(HIST):
        lane = (h % 2) * EMB_DIM
        part = w3[:, h // 2, lane : lane + EMB_DIM]
        out_ref[h] = jnp.transpose(part, (1, 0))


def _out_transpose(o2):
    return pl.pallas_call(
        _ot_body,
        grid=(NBB,),
        in_specs=[pl.BlockSpec((RPB, 128), lambda i: (i, 0))],
        out_specs=pl.BlockSpec((HIST, EMB_DIM, BLK_B), lambda i: (0, 0, i)),
        out_shape=jax.ShapeDtypeStruct((HIST, EMB_DIM, BATCH), jnp.float32),
    )(o2)


def _emb_body(idx_hbm, table_hbm, out_hbm, idx_v, rows_v, sems):
    wid = lax.axis_index("s") * NC + lax.axis_index("c")
    base = wid * B_PER_W

    pltpu.sync_copy(idx_hbm.at[wid], idx_v)

    def remap(i, carry):
        r = i // (CHUNK // 16)
        c = (i % (CHUNK // 16)) * 16
        v = idx_v[r, pl.ds(c, 16)]
        rp = v & (VB - 1)
        lo = v + rp
        row = jnp.where(rp < VB // 2, lo, lo - (VB - 1))
        idx_v[r, pl.ds(c, 16)] = row
        return carry

    lax.fori_loop(0, N_CHUNKS * (CHUNK // 16), remap, 0)

    def start(b, j):
        pltpu.async_copy(table_hbm.at[idx_v.at[j]], rows_v.at[b], sems.at[b])

    def finish(b, j):
        pltpu.make_async_copy(
            table_hbm.at[idx_v.at[j]], rows_v.at[b], sems.at[b]
        ).wait()
        pltpu.sync_copy(rows_v.at[b], out_hbm.at[pl.ds(base + j * CHUNK, CHUNK)])

    for b in range(NBUF):
        start(b, b)

    def body(it, carry):
        for b in range(NBUF):
            j = it * NBUF + b
            finish(b, j)
            start(b, j + NBUF)
        return carry

    lax.fori_loop(0, T - 1, body, 0)

    for b in range(NBUF):
        finish(b, (T - 1) * NBUF + b)


@jax.jit
def _emb_call(x_flat, table):
    mesh = plsc.VectorSubcoreMesh(core_axis_name="c", subcore_axis_name="s")
    kern = pl.kernel(
        _emb_body,
        out_type=jax.ShapeDtypeStruct((B, EMB_DIM), jnp.float32),
        mesh=mesh,
        scratch_types=[
            pltpu.VMEM((N_CHUNKS, CHUNK), jnp.int32),
            pltpu.VMEM((NBUF, CHUNK, EMB_DIM), jnp.float32),
            pltpu.SemaphoreType.DMA((NBUF,)),
        ],
        compiler_params=pltpu.CompilerParams(use_tc_tiling_on_sc=False),
    )
    return kern(x_flat, table)


def kernel(x, weight):
    w2 = _transpose_tc(weight.T)
    table = w2.reshape(N2, EMB_DIM)
    x_flat = x.reshape(NW, N_CHUNKS, CHUNK).astype(jnp.int32)
    out = _emb_call(x_flat, table)
    o2 = out.reshape(B * EMB_DIM // 128, 128)
    outT = _out_transpose(o2)
    return jnp.transpose(outT, (2, 0, 1))


# speedup vs baseline: 1.0831x; 1.0804x over previous
import jax
import jax.numpy as jnp
from jax import lax
from jax.experimental import pallas as pl
from jax.experimental.pallas import tpu as pltpu
from jax.experimental.pallas import tpu_sc as plsc

N_TOKEN = 1000000
EMB_DIM = 64
BATCH = 16384
HIST = 20

NC = 2
NS = 16
NW = NC * NS

B = BATCH * HIST
B_PER_W = B // NW
CHUNK = 128
N_CHUNKS = B_PER_W // CHUNK
NBUF = 8
T = N_CHUNKS // NBUF

VB = 32768
NVB = (N_TOKEN + VB - 1) // VB
N2 = NVB * VB


def _tr_body(wt_ref, out_ref):
    blk = wt_ref[...]
    tr = jnp.transpose(blk, (1, 0))
    out_ref[:, 0:EMB_DIM] = tr[: VB // 2, :]
    out_ref[:, EMB_DIM:128] = tr[VB // 2 :, :]


def _transpose_tc(wT):
    return pl.pallas_call(
        _tr_body,
        grid=(NVB,),
        in_specs=[pl.BlockSpec((EMB_DIM, VB), lambda j: (0, j))],
        out_specs=pl.BlockSpec((VB // 2, 128), lambda j: (j, 0)),
        out_shape=jax.ShapeDtypeStruct((N2 // 2, 128), jnp.float32),
    )(wT)


BLK_B = 512
NBB = BATCH // BLK_B
RPB = BLK_B * HIST * EMB_DIM // 128


def _ot_body(in_ref, out_ref):
    whole = in_ref[...]
    w3 = whole.reshape(BLK_B, HIST * EMB_DIM // 128, 128)
    for k in range(HIST // 2):
        t = jnp.transpose(w3[:, k, :], (1, 0))
        out_ref[2 * k] = t[:EMB_DIM]
        out_ref[2 * k + 1] = t[EMB_DIM:]


def _out_transpose(o2):
    return pl.pallas_call(
        _ot_body,
        grid=(NBB,),
        in_specs=[pl.BlockSpec((RPB, 128), lambda i: (i, 0))],
        out_specs=pl.BlockSpec((HIST, EMB_DIM, BLK_B), lambda i: (0, 0, i)),
        out_shape=jax.ShapeDtypeStruct((HIST, EMB_DIM, BATCH), jnp.float32),
    )(o2)


def _emb_body(idx_hbm, table_hbm, out_hbm, idx_v, rows_v, sems):
    wid = lax.axis_index("s") * NC + lax.axis_index("c")
    base = wid * B_PER_W

    pltpu.sync_copy(idx_hbm.at[wid], idx_v)

    def remap(i, carry):
        r = i // (CHUNK // 16)
        c = (i % (CHUNK // 16)) * 16
        v = idx_v[r, pl.ds(c, 16)]
        rp = v & (VB - 1)
        lo = v + rp
        row = jnp.where(rp < VB // 2, lo, lo - (VB - 1))
        idx_v[r, pl.ds(c, 16)] = row
        return carry

    lax.fori_loop(0, N_CHUNKS * (CHUNK // 16), remap, 0)

    def start(b, j):
        pltpu.async_copy(table_hbm.at[idx_v.at[j]], rows_v.at[b], sems.at[b])

    def finish(b, j):
        pltpu.make_async_copy(
            table_hbm.at[idx_v.at[j]], rows_v.at[b], sems.at[b]
        ).wait()
        pltpu.sync_copy(rows_v.at[b], out_hbm.at[pl.ds(base + j * CHUNK, CHUNK)])

    for b in range(NBUF):
        start(b, b)

    def body(it, carry):
        for b in range(NBUF):
            j = it * NBUF + b
            finish(b, j)
            start(b, j + NBUF)
        return carry

    lax.fori_loop(0, T - 1, body, 0)

    for b in range(NBUF):
        finish(b, (T - 1) * NBUF + b)


@jax.jit
def _emb_call(x_flat, table):
    mesh = plsc.VectorSubcoreMesh(core_axis_name="c", subcore_axis_name="s")
    kern = pl.kernel(
        _emb_body,
        out_type=jax.ShapeDtypeStruct((B, EMB_DIM), jnp.float32),
        mesh=mesh,
        scratch_types=[
            pltpu.VMEM((N_CHUNKS, CHUNK), jnp.int32),
            pltpu.VMEM((NBUF, CHUNK, EMB_DIM), jnp.float32),
            pltpu.SemaphoreType.DMA((NBUF,)),
        ],
        compiler_params=pltpu.CompilerParams(use_tc_tiling_on_sc=False),
    )
    return kern(x_flat, table)


def kernel(x, weight):
    w2 = _transpose_tc(weight.T)
    table = w2.reshape(N2, EMB_DIM)
    x_flat = x.reshape(NW, N_CHUNKS, CHUNK).astype(jnp.int32)
    out = _emb_call(x_flat, table)
    o2 = out.reshape(B * EMB_DIM // 128, 128)
    outT = _out_transpose(o2)
    return jnp.transpose(outT, (2, 0, 1))


# speedup vs baseline: 1.2362x; 1.1414x over previous
import jax
import jax.numpy as jnp
from jax import lax
from jax.experimental import pallas as pl
from jax.experimental.pallas import tpu as pltpu
from jax.experimental.pallas import tpu_sc as plsc

N_TOKEN = 1000000
EMB_DIM = 64
BATCH = 16384
HIST = 20

NC = 2
NS = 16
NW = NC * NS

B = BATCH * HIST
B_PER_W = B // NW
CHUNK = 128
N_CHUNKS = B_PER_W // CHUNK
NBUF = 8
T = N_CHUNKS // NBUF

VB = 32768
NVB = (N_TOKEN + VB - 1) // VB
N2 = NVB * VB


def _tr_body(wt_ref, out_ref):
    lo = wt_ref[:, : VB // 2]
    hi = wt_ref[:, VB // 2 :]
    stacked = jnp.concatenate([lo, hi], axis=0)
    out_ref[...] = jnp.transpose(stacked, (1, 0))


def _transpose_tc(wT):
    return pl.pallas_call(
        _tr_body,
        grid=(NVB,),
        in_specs=[pl.BlockSpec((EMB_DIM, VB), lambda j: (0, j))],
        out_specs=pl.BlockSpec((VB // 2, 128), lambda j: (j, 0)),
        out_shape=jax.ShapeDtypeStruct((N2 // 2, 128), jnp.float32),
    )(wT)


BLK_B = 512
NBB = BATCH // BLK_B
RPB = BLK_B * HIST * EMB_DIM // 128


def _ot_body(in_ref, out_ref):
    whole = in_ref[...]
    w3 = whole.reshape(BLK_B, HIST * EMB_DIM // 128, 128)
    for k in range(HIST // 2):
        t = jnp.transpose(w3[:, k, :], (1, 0))
        out_ref[2 * k] = t[:EMB_DIM]
        out_ref[2 * k + 1] = t[EMB_DIM:]


def _out_transpose(o2):
    return pl.pallas_call(
        _ot_body,
        grid=(NBB,),
        in_specs=[pl.BlockSpec((RPB, 128), lambda i: (i, 0))],
        out_specs=pl.BlockSpec((HIST, EMB_DIM, BLK_B), lambda i: (0, 0, i)),
        out_shape=jax.ShapeDtypeStruct((HIST, EMB_DIM, BATCH), jnp.float32),
    )(o2)


def _emb_body(idx_hbm, table_hbm, out_hbm, idx_v, rows_v, sems):
    wid = lax.axis_index("s") * NC + lax.axis_index("c")
    base = wid * B_PER_W

    pltpu.sync_copy(idx_hbm.at[wid], idx_v)

    def remap(i, carry):
        r = i // (CHUNK // 16)
        c = (i % (CHUNK // 16)) * 16
        v = idx_v[r, pl.ds(c, 16)]
        rp = v & (VB - 1)
        lo = v + rp
        row = jnp.where(rp < VB // 2, lo, lo - (VB - 1))
        idx_v[r, pl.ds(c, 16)] = row
        return carry

    lax.fori_loop(0, N_CHUNKS * (CHUNK // 16), remap, 0)

    def start(b, j):
        pltpu.async_copy(table_hbm.at[idx_v.at[j]], rows_v.at[b], sems.at[b])

    def finish(b, j):
        pltpu.make_async_copy(
            table_hbm.at[idx_v.at[j]], rows_v.at[b], sems.at[b]
        ).wait()
        pltpu.sync_copy(rows_v.at[b], out_hbm.at[pl.ds(base + j * CHUNK, CHUNK)])

    for b in range(NBUF):
        start(b, b)

    def body(it, carry):
        for b in range(NBUF):
            j = it * NBUF + b
            finish(b, j)
            start(b, j + NBUF)
        return carry

    lax.fori_loop(0, T - 1, body, 0)

    for b in range(NBUF):
        finish(b, (T - 1) * NBUF + b)


@jax.jit
def _emb_call(x_flat, table):
    mesh = plsc.VectorSubcoreMesh(core_axis_name="c", subcore_axis_name="s")
    kern = pl.kernel(
        _emb_body,
        out_type=jax.ShapeDtypeStruct((B, EMB_DIM), jnp.float32),
        mesh=mesh,
        scratch_types=[
            pltpu.VMEM((N_CHUNKS, CHUNK), jnp.int32),
            pltpu.VMEM((NBUF, CHUNK, EMB_DIM), jnp.float32),
            pltpu.SemaphoreType.DMA((NBUF,)),
        ],
        compiler_params=pltpu.CompilerParams(use_tc_tiling_on_sc=False),
    )
    return kern(x_flat, table)


def kernel(x, weight):
    w2 = _transpose_tc(weight.T)
    table = w2.reshape(N2, EMB_DIM)
    x_flat = x.reshape(NW, N_CHUNKS, CHUNK).astype(jnp.int32)
    out = _emb_call(x_flat, table)
    o2 = out.reshape(B * EMB_DIM // 128, 128)
    outT = _out_transpose(o2)
    return jnp.transpose(outT, (2, 0, 1))


# speedup vs baseline: 1.2655x; 1.0237x over previous
import jax
import jax.numpy as jnp
from jax import lax
from jax.experimental import pallas as pl
from jax.experimental.pallas import tpu as pltpu
from jax.experimental.pallas import tpu_sc as plsc

N_TOKEN = 1000000
EMB_DIM = 64
BATCH = 16384
HIST = 20

NC = 2
NS = 16
NW = NC * NS

B = BATCH * HIST
B_PER_W = B // NW
CHUNK = 128
N_CHUNKS = B_PER_W // CHUNK
NBUF = 8
T = N_CHUNKS // NBUF

VB = 32768
NVB = (N_TOKEN + VB - 1) // VB
N2 = NVB * VB


def _tr_body(wt_ref, out_ref):
    lo = wt_ref[:, : VB // 2]
    hi = wt_ref[:, VB // 2 :]
    stacked = jnp.concatenate([lo, hi], axis=0)
    out_ref[...] = jnp.transpose(stacked, (1, 0))


def _transpose_tc(wT):
    return pl.pallas_call(
        _tr_body,
        grid=(NVB,),
        in_specs=[pl.BlockSpec((EMB_DIM, VB), lambda j: (0, j))],
        out_specs=pl.BlockSpec((VB // 2, 128), lambda j: (j, 0)),
        out_shape=jax.ShapeDtypeStruct((N2 // 2, 128), jnp.float32),
    )(wT)


BLK_B = 1024
NBB = BATCH // BLK_B
RPB = BLK_B * HIST * EMB_DIM // 128


def _ot_body(in_ref, out_ref):
    whole = in_ref[...]
    w3 = whole.reshape(BLK_B, HIST * EMB_DIM // 128, 128)
    for k in range(HIST // 2):
        t = jnp.transpose(w3[:, k, :], (1, 0))
        out_ref[2 * k] = t[:EMB_DIM]
        out_ref[2 * k + 1] = t[EMB_DIM:]


def _out_transpose(o2):
    return pl.pallas_call(
        _ot_body,
        grid=(NBB,),
        in_specs=[pl.BlockSpec((RPB, 128), lambda i: (i, 0))],
        out_specs=pl.BlockSpec((HIST, EMB_DIM, BLK_B), lambda i: (0, 0, i)),
        out_shape=jax.ShapeDtypeStruct((HIST, EMB_DIM, BATCH), jnp.float32),
    )(o2)


def _emb_body(idx_hbm, table_hbm, out_hbm, idx_v, rows_v, sems):
    wid = lax.axis_index("s") * NC + lax.axis_index("c")
    base = wid * B_PER_W

    pltpu.sync_copy(idx_hbm.at[wid], idx_v)

    def remap(i, carry):
        r = i // (CHUNK // 16)
        c = (i % (CHUNK // 16)) * 16
        v = idx_v[r, pl.ds(c, 16)]
        rp = v & (VB - 1)
        lo = v + rp
        row = jnp.where(rp < VB // 2, lo, lo - (VB - 1))
        idx_v[r, pl.ds(c, 16)] = row
        return carry

    lax.fori_loop(0, N_CHUNKS * (CHUNK // 16), remap, 0)

    def start(b, j):
        pltpu.async_copy(table_hbm.at[idx_v.at[j]], rows_v.at[b], sems.at[b])

    def finish(b, j):
        pltpu.make_async_copy(
            table_hbm.at[idx_v.at[j]], rows_v.at[b], sems.at[b]
        ).wait()
        pltpu.sync_copy(rows_v.at[b], out_hbm.at[pl.ds(base + j * CHUNK, CHUNK)])

    for b in range(NBUF):
        start(b, b)

    def body(it, carry):
        for b in range(NBUF):
            j = it * NBUF + b
            finish(b, j)
            start(b, j + NBUF)
        return carry

    lax.fori_loop(0, T - 1, body, 0)

    for b in range(NBUF):
        finish(b, (T - 1) * NBUF + b)


@jax.jit
def _emb_call(x_flat, table):
    mesh = plsc.VectorSubcoreMesh(core_axis_name="c", subcore_axis_name="s")
    kern = pl.kernel(
        _emb_body,
        out_type=jax.ShapeDtypeStruct((B, EMB_DIM), jnp.float32),
        mesh=mesh,
        scratch_types=[
            pltpu.VMEM((N_CHUNKS, CHUNK), jnp.int32),
            pltpu.VMEM((NBUF, CHUNK, EMB_DIM), jnp.float32),
            pltpu.SemaphoreType.DMA((NBUF,)),
        ],
        compiler_params=pltpu.CompilerParams(use_tc_tiling_on_sc=False),
    )
    return kern(x_flat, table)


def kernel(x, weight):
    w2 = _transpose_tc(weight.T)
    table = w2.reshape(N2, EMB_DIM)
    x_flat = x.reshape(NW, N_CHUNKS, CHUNK).astype(jnp.int32)
    out = _emb_call(x_flat, table)
    o2 = out.reshape(B * EMB_DIM // 128, 128)
    outT = _out_transpose(o2)
    return jnp.transpose(outT, (2, 0, 1))


# speedup vs baseline: 1.2660x; 1.0004x over previous
import jax
import jax.numpy as jnp
from jax import lax
from jax.experimental import pallas as pl
from jax.experimental.pallas import tpu as pltpu
from jax.experimental.pallas import tpu_sc as plsc

N_TOKEN = 1000000
EMB_DIM = 64
BATCH = 16384
HIST = 20

NC = 2
NS = 16
NW = NC * NS

B = BATCH * HIST
B_PER_W = B // NW
CHUNK = 128
N_CHUNKS = B_PER_W // CHUNK
NBUF = 8
T = N_CHUNKS // NBUF

VB = 32768
NVB = (N_TOKEN + VB - 1) // VB
N2 = NVB * VB


def _tr_body(wt_ref, out_ref):
    lo = wt_ref[:, : VB // 2]
    hi = wt_ref[:, VB // 2 :]
    stacked = jnp.concatenate([lo, hi], axis=0)
    out_ref[...] = jnp.transpose(stacked, (1, 0))


def _transpose_tc(wT):
    return pl.pallas_call(
        _tr_body,
        grid=(NVB,),
        in_specs=[pl.BlockSpec((EMB_DIM, VB), lambda j: (0, j))],
        out_specs=pl.BlockSpec((VB // 2, 128), lambda j: (j, 0)),
        out_shape=jax.ShapeDtypeStruct((N2 // 2, 128), jnp.float32),
    )(wT)


BLK_B = 2048
NBB = BATCH // BLK_B
RPB = BLK_B * HIST * EMB_DIM // 128


def _ot_body(in_ref, out_ref):
    whole = in_ref[...]
    w3 = whole.reshape(BLK_B, HIST * EMB_DIM // 128, 128)
    for k in range(HIST // 2):
        t = jnp.transpose(w3[:, k, :], (1, 0))
        out_ref[2 * k] = t[:EMB_DIM]
        out_ref[2 * k + 1] = t[EMB_DIM:]


def _out_transpose(o2):
    return pl.pallas_call(
        _ot_body,
        grid=(NBB,),
        in_specs=[pl.BlockSpec((RPB, 128), lambda i: (i, 0))],
        out_specs=pl.BlockSpec((HIST, EMB_DIM, BLK_B), lambda i: (0, 0, i)),
        out_shape=jax.ShapeDtypeStruct((HIST, EMB_DIM, BATCH), jnp.float32),
    )(o2)


def _emb_body(idx_hbm, table_hbm, out_hbm, idx_v, rows_v, sems):
    wid = lax.axis_index("s") * NC + lax.axis_index("c")
    base = wid * B_PER_W

    pltpu.sync_copy(idx_hbm.at[wid], idx_v)

    def remap(i, carry):
        r = i // (CHUNK // 16)
        c = (i % (CHUNK // 16)) * 16
        v = idx_v[r, pl.ds(c, 16)]
        rp = v & (VB - 1)
        lo = v + rp
        row = jnp.where(rp < VB // 2, lo, lo - (VB - 1))
        idx_v[r, pl.ds(c, 16)] = row
        return carry

    lax.fori_loop(0, N_CHUNKS * (CHUNK // 16), remap, 0)

    def start(b, j):
        pltpu.async_copy(table_hbm.at[idx_v.at[j]], rows_v.at[b], sems.at[b])

    def finish(b, j):
        pltpu.make_async_copy(
            table_hbm.at[idx_v.at[j]], rows_v.at[b], sems.at[b]
        ).wait()
        pltpu.sync_copy(rows_v.at[b], out_hbm.at[pl.ds(base + j * CHUNK, CHUNK)])

    for b in range(NBUF):
        start(b, b)

    def body(it, carry):
        for b in range(NBUF):
            j = it * NBUF + b
            finish(b, j)
            start(b, j + NBUF)
        return carry

    lax.fori_loop(0, T - 1, body, 0)

    for b in range(NBUF):
        finish(b, (T - 1) * NBUF + b)


@jax.jit
def _emb_call(x_flat, table):
    mesh = plsc.VectorSubcoreMesh(core_axis_name="c", subcore_axis_name="s")
    kern = pl.kernel(
        _emb_body,
        out_type=jax.ShapeDtypeStruct((B, EMB_DIM), jnp.float32),
        mesh=mesh,
        scratch_types=[
            pltpu.VMEM((N_CHUNKS, CHUNK), jnp.int32),
            pltpu.VMEM((NBUF, CHUNK, EMB_DIM), jnp.float32),
            pltpu.SemaphoreType.DMA((NBUF,)),
        ],
        compiler_params=pltpu.CompilerParams(use_tc_tiling_on_sc=False),
    )
    return kern(x_flat, table)


def kernel(x, weight):
    w2 = _transpose_tc(weight.T)
    table = w2.reshape(N2, EMB_DIM)
    x_flat = x.reshape(NW, N_CHUNKS, CHUNK).astype(jnp.int32)
    out = _emb_call(x_flat, table)
    o2 = out.reshape(B * EMB_DIM // 128, 128)
    outT = _out_transpose(o2)
    return jnp.transpose(outT, (2, 0, 1))
